# baseline (speedup 1.0000x reference)
import functools

import jax
import jax.numpy as jnp
from jax import lax
from jax.experimental import pallas as pl
from jax.experimental.pallas import tpu as pltpu
from jax.experimental.pallas import tpu_sc as plsc

B = 4096
DIM = 1024
K = 65536
TOP_N = 10
PAD = 16

BK = 256
NK = K // BK
BB = 512
NBB = B // BB


def _topk_body(q_ref, queue_ref, idx_out_ref, xn_ref, runv_ref, runi_ref):
    k = pl.program_id(0)

    @pl.when(k == 0)
    def _init():
        def b_init(b, carry):
            off = b * BB
            xq = q_ref[pl.ds(off, BB), :]
            n = jnp.sqrt(jnp.sum(xq * xq, axis=1, keepdims=True))
            xn_ref[pl.ds(off, BB), :] = xq / jnp.maximum(n, 1e-12)
            runv_ref[pl.ds(off, BB), :] = jnp.full(
                (BB, PAD), -jnp.inf, dtype=jnp.float32)
            runi_ref[pl.ds(off, BB), :] = jnp.zeros((BB, PAD), dtype=jnp.int32)
            return carry
        lax.fori_loop(0, NBB, b_init, 0)

    qb = queue_ref[...]
    qn = qb / jnp.maximum(
        jnp.sqrt(jnp.sum(qb * qb, axis=1, keepdims=True)), 1e-12)
    colid = lax.broadcasted_iota(jnp.int32, (BB, BK), 1) + k * BK
    slot = lax.broadcasted_iota(jnp.int32, (BB, PAD), 1)

    def b_step(b, carry):
        off = b * BB
        s = lax.dot_general(
            xn_ref[pl.ds(off, BB), :].astype(jnp.bfloat16),
            qn.astype(jnp.bfloat16),
            dimension_numbers=(((1,), (1,)), ((), ())),
            preferred_element_type=jnp.float32,
        )
        runv = runv_ref[pl.ds(off, BB), :]
        runi = runi_ref[pl.ds(off, BB), :]
        for _ in range(TOP_N):
            m = jnp.max(s, axis=1, keepdims=True)
            am = jnp.min(jnp.where(s == m, colid, K), axis=1, keepdims=True)
            s = jnp.where(colid == am, -jnp.inf, s)
            pos = jnp.sum((runv >= m).astype(jnp.int32), axis=1, keepdims=True)
            shv = jnp.concatenate([runv[:, :1], runv[:, :-1]], axis=1)
            shi = jnp.concatenate([runi[:, :1], runi[:, :-1]], axis=1)
            runv = jnp.where(slot < pos, runv, jnp.where(slot == pos, m, shv))
            runi = jnp.where(slot < pos, runi, jnp.where(slot == pos, am, shi))
        runv_ref[pl.ds(off, BB), :] = runv
        runi_ref[pl.ds(off, BB), :] = runi
        return carry

    lax.fori_loop(0, NBB, b_step, 0)

    @pl.when(k == NK - 1)
    def _emit():
        idx_out_ref[...] = runi_ref[...]


def _topk_indices(query, queue_q):
    return pl.pallas_call(
        _topk_body,
        grid=(NK,),
        in_specs=[
            pl.BlockSpec((B, DIM), lambda k: (0, 0)),
            pl.BlockSpec((BK, DIM), lambda k: (k, 0)),
        ],
        out_specs=pl.BlockSpec((B, PAD), lambda k: (0, 0)),
        out_shape=jax.ShapeDtypeStruct((B, PAD), jnp.int32),
        scratch_shapes=[
            pltpu.VMEM((B, DIM), jnp.float32),
            pltpu.VMEM((B, PAD), jnp.float32),
            pltpu.VMEM((B, PAD), jnp.int32),
        ],
        compiler_params=pltpu.CompilerParams(
            dimension_semantics=("arbitrary",),
        ),
    )(query, queue_q)


NW = 32
NB_ROWS = B * TOP_N
B_PER_W = NB_ROWS // NW
CH = 64
NCH = B_PER_W // CH


def _gather_body(table_hbm, idx_hbm, out_hbm, idx_v, rows_v, sem):
    wid = lax.axis_index("s") * 2 + lax.axis_index("c")
    base = wid * B_PER_W
    pltpu.sync_copy(idx_hbm.at[pl.ds(base, B_PER_W)], idx_v)

    def chunk(c, carry):
        off = c * CH
        pltpu.async_copy(
            table_hbm.at[idx_v.at[pl.ds(off, CH)]], rows_v, sem).wait()
        pltpu.sync_copy(rows_v, out_hbm.at[pl.ds(base + off, CH)])
        return carry

    lax.fori_loop(0, NCH, chunk, 0)


def _gather_rows(queue_q, flat_idx):
    mesh = plsc.VectorSubcoreMesh(core_axis_name="c", subcore_axis_name="s")
    f = functools.partial(
        pl.kernel,
        mesh=mesh,
        out_type=jax.ShapeDtypeStruct((NB_ROWS, DIM), jnp.float32),
        scratch_types=[
            pltpu.VMEM((B_PER_W,), jnp.int32),
            pltpu.VMEM((CH, DIM), jnp.float32),
            pltpu.SemaphoreType.DMA,
        ],
    )(_gather_body)
    return f(queue_q, flat_idx)


def kernel(query, queue_q):
    idx = _topk_indices(query, queue_q)
    flat_idx = idx[:, :TOP_N].reshape(-1)
    rows = _gather_rows(queue_q, flat_idx)
    return rows.reshape(B, TOP_N, DIM)


# speedup vs baseline: 2.0525x; 2.0525x over previous
import functools

import jax
import jax.numpy as jnp
from jax import lax
from jax.experimental import pallas as pl
from jax.experimental.pallas import tpu as pltpu
from jax.experimental.pallas import tpu_sc as plsc

B = 4096
DIM = 1024
K = 65536
TOP_N = 10
PAD = 16

BK = 256
NK = K // BK
BB = 512
NBB = B // BB


def _topk_body(q_ref, queue_ref, qnorm_ref, tnorm_ref, idx_out_ref,
               xn_ref, runv_ref, runi_ref):
    k = pl.program_id(0)

    @pl.when(k == 0)
    def _init():
        def b_init(b, carry):
            off = b * BB
            xq = q_ref[pl.ds(off, BB), :]
            n = qnorm_ref[pl.ds(off, BB), :]
            xn_ref[pl.ds(off, BB), :] = (
                xq / jnp.maximum(n, 1e-12)).astype(jnp.bfloat16)
            runv_ref[pl.ds(off, BB), :] = jnp.full(
                (BB, PAD), -jnp.inf, dtype=jnp.float32)
            runi_ref[pl.ds(off, BB), :] = jnp.zeros((BB, PAD), dtype=jnp.int32)
            return carry
        lax.fori_loop(0, NBB, b_init, 0)

    qb = queue_ref[...]
    qn = (qb / jnp.maximum(tnorm_ref[...], 1e-12)).astype(jnp.bfloat16)
    colid = lax.broadcasted_iota(jnp.int32, (BB, BK), 1) + k * BK
    slot = lax.broadcasted_iota(jnp.int32, (BB, PAD), 1)

    def b_step(b, carry):
        off = b * BB
        s0 = lax.dot_general(
            xn_ref[pl.ds(off, BB), :], qn,
            dimension_numbers=(((1,), (1,)), ((), ())),
            preferred_element_type=jnp.float32,
        )
        runv0 = runv_ref[pl.ds(off, BB), :]
        runi0 = runi_ref[pl.ds(off, BB), :]

        def cond(c):
            _, m, runv, _ = c
            return jnp.any(m > runv[:, TOP_N - 1:TOP_N])

        def body(c):
            s, m, runv, runi = c
            am = jnp.min(jnp.where(s == m, colid, K), axis=1, keepdims=True)
            s = jnp.where(colid == am, -jnp.inf, s)
            pos = jnp.sum((runv >= m).astype(jnp.int32), axis=1, keepdims=True)
            shv = jnp.concatenate([runv[:, :1], runv[:, :-1]], axis=1)
            shi = jnp.concatenate([runi[:, :1], runi[:, :-1]], axis=1)
            runv = jnp.where(slot < pos, runv, jnp.where(slot == pos, m, shv))
            runi = jnp.where(slot < pos, runi, jnp.where(slot == pos, am, shi))
            return s, jnp.max(s, axis=1, keepdims=True), runv, runi

        m0 = jnp.max(s0, axis=1, keepdims=True)
        _, _, runv, runi = lax.while_loop(
            cond, body, (s0, m0, runv0, runi0))
        runv_ref[pl.ds(off, BB), :] = runv
        runi_ref[pl.ds(off, BB), :] = runi
        return carry

    lax.fori_loop(0, NBB, b_step, 0)

    @pl.when(k == NK - 1)
    def _emit():
        idx_out_ref[...] = runi_ref[...]


def _topk_indices(query, queue_q, qnorm, tnorm):
    return pl.pallas_call(
        _topk_body,
        grid=(NK,),
        in_specs=[
            pl.BlockSpec((B, DIM), lambda k: (0, 0)),
            pl.BlockSpec((BK, DIM), lambda k: (k, 0)),
            pl.BlockSpec((B, 1), lambda k: (0, 0)),
            pl.BlockSpec((BK, 1), lambda k: (k, 0)),
        ],
        out_specs=pl.BlockSpec((B, PAD), lambda k: (0, 0)),
        out_shape=jax.ShapeDtypeStruct((B, PAD), jnp.int32),
        scratch_shapes=[
            pltpu.VMEM((B, DIM), jnp.bfloat16),
            pltpu.VMEM((B, PAD), jnp.float32),
            pltpu.VMEM((B, PAD), jnp.int32),
        ],
        compiler_params=pltpu.CompilerParams(
            dimension_semantics=("arbitrary",),
        ),
    )(query, queue_q, qnorm, tnorm)


NW = 32
NB_ROWS = B * TOP_N
B_PER_W = NB_ROWS // NW
CH = 64
NCH = B_PER_W // CH


def _gather_body(table_hbm, idx_hbm, out_hbm, idx_v, rows_v, sem):
    wid = lax.axis_index("s") * 2 + lax.axis_index("c")
    base = wid * B_PER_W
    pltpu.sync_copy(idx_hbm.at[pl.ds(base, B_PER_W)], idx_v)

    def chunk(c, carry):
        off = c * CH
        pltpu.async_copy(
            table_hbm.at[idx_v.at[pl.ds(off, CH)]], rows_v, sem).wait()
        pltpu.sync_copy(rows_v, out_hbm.at[pl.ds(base + off, CH)])
        return carry

    lax.fori_loop(0, NCH, chunk, 0)


def _gather_rows(queue_q, flat_idx):
    mesh = plsc.VectorSubcoreMesh(core_axis_name="c", subcore_axis_name="s")
    f = functools.partial(
        pl.kernel,
        mesh=mesh,
        out_type=jax.ShapeDtypeStruct((NB_ROWS, DIM), jnp.float32),
        scratch_types=[
            pltpu.VMEM((B_PER_W,), jnp.int32),
            pltpu.VMEM((CH, DIM), jnp.float32),
            pltpu.SemaphoreType.DMA,
        ],
    )(_gather_body)
    return f(queue_q, flat_idx)


def kernel(query, queue_q):
    qnorm = jnp.linalg.norm(query, ord=2, axis=-1, keepdims=True)
    tnorm = jnp.linalg.norm(queue_q, ord=2, axis=-1, keepdims=True)
    idx = _topk_indices(query, queue_q, qnorm, tnorm)
    flat_idx = idx[:, :TOP_N].reshape(-1)
    rows = _gather_rows(queue_q, flat_idx)
    return rows.reshape(B, TOP_N, DIM)


# speedup vs baseline: 2.0706x; 1.0088x over previous
import functools

import jax
import jax.numpy as jnp
from jax import lax
from jax.experimental import pallas as pl
from jax.experimental.pallas import tpu as pltpu
from jax.experimental.pallas import tpu_sc as plsc

B = 4096
DIM = 1024
K = 65536
TOP_N = 10
PAD = 16

BK = 256
NK = K // BK
BB = 512
NBB = B // BB


def _topk_body(q_ref, queue_ref, qnorm_ref, tnorm_ref, idx_out_ref,
               xn_ref, runv_ref, runi_ref):
    k = pl.program_id(0)

    @pl.when(k == 0)
    def _init():
        def b_init(b, carry):
            off = b * BB
            xq = q_ref[pl.ds(off, BB), :]
            n = qnorm_ref[pl.ds(off, BB), :]
            xn_ref[pl.ds(off, BB), :] = xq / jnp.maximum(n, 1e-12)
            runv_ref[pl.ds(off, BB), :] = jnp.full(
                (BB, PAD), -jnp.inf, dtype=jnp.float32)
            runi_ref[pl.ds(off, BB), :] = jnp.zeros((BB, PAD), dtype=jnp.int32)
            return carry
        lax.fori_loop(0, NBB, b_init, 0)

    qb = queue_ref[...]
    qn = qb / jnp.maximum(tnorm_ref[...], 1e-12)
    colid = lax.broadcasted_iota(jnp.int32, (BB, BK), 1) + k * BK
    slot = lax.broadcasted_iota(jnp.int32, (BB, PAD), 1)

    def b_step(b, carry):
        off = b * BB
        s0 = lax.dot_general(
            xn_ref[pl.ds(off, BB), :], qn,
            dimension_numbers=(((1,), (1,)), ((), ())),
            preferred_element_type=jnp.float32,
        )
        runv0 = runv_ref[pl.ds(off, BB), :]
        runi0 = runi_ref[pl.ds(off, BB), :]

        def cond(c):
            _, m, runv, _ = c
            return jnp.any(m > runv[:, TOP_N - 1:TOP_N])

        def body(c):
            s, m, runv, runi = c
            am = jnp.min(jnp.where(s == m, colid, K), axis=1, keepdims=True)
            s = jnp.where(colid == am, -jnp.inf, s)
            pos = jnp.sum((runv >= m).astype(jnp.int32), axis=1, keepdims=True)
            shv = jnp.concatenate([runv[:, :1], runv[:, :-1]], axis=1)
            shi = jnp.concatenate([runi[:, :1], runi[:, :-1]], axis=1)
            runv = jnp.where(slot < pos, runv, jnp.where(slot == pos, m, shv))
            runi = jnp.where(slot < pos, runi, jnp.where(slot == pos, am, shi))
            return s, jnp.max(s, axis=1, keepdims=True), runv, runi

        m0 = jnp.max(s0, axis=1, keepdims=True)
        _, _, runv, runi = lax.while_loop(
            cond, body, (s0, m0, runv0, runi0))
        runv_ref[pl.ds(off, BB), :] = runv
        runi_ref[pl.ds(off, BB), :] = runi
        return carry

    lax.fori_loop(0, NBB, b_step, 0)

    @pl.when(k == NK - 1)
    def _emit():
        idx_out_ref[...] = runi_ref[...]


def _topk_indices(query, queue_q, qnorm, tnorm):
    return pl.pallas_call(
        _topk_body,
        grid=(NK,),
        in_specs=[
            pl.BlockSpec((B, DIM), lambda k: (0, 0)),
            pl.BlockSpec((BK, DIM), lambda k: (k, 0)),
            pl.BlockSpec((B, 1), lambda k: (0, 0)),
            pl.BlockSpec((BK, 1), lambda k: (k, 0)),
        ],
        out_specs=pl.BlockSpec((B, PAD), lambda k: (0, 0)),
        out_shape=jax.ShapeDtypeStruct((B, PAD), jnp.int32),
        scratch_shapes=[
            pltpu.VMEM((B, DIM), jnp.float32),
            pltpu.VMEM((B, PAD), jnp.float32),
            pltpu.VMEM((B, PAD), jnp.int32),
        ],
        compiler_params=pltpu.CompilerParams(
            dimension_semantics=("arbitrary",),
        ),
    )(query, queue_q, qnorm, tnorm)


NW = 32
NB_ROWS = B * TOP_N
B_PER_W = NB_ROWS // NW
CH = 64
NCH = B_PER_W // CH


def _gather_body(table_hbm, idx_hbm, out_hbm, idx_v, rows_v, sem):
    wid = lax.axis_index("s") * 2 + lax.axis_index("c")
    base = wid * B_PER_W
    pltpu.sync_copy(idx_hbm.at[pl.ds(base, B_PER_W)], idx_v)

    def chunk(c, carry):
        off = c * CH
        pltpu.async_copy(
            table_hbm.at[idx_v.at[pl.ds(off, CH)]], rows_v, sem).wait()
        pltpu.sync_copy(rows_v, out_hbm.at[pl.ds(base + off, CH)])
        return carry

    lax.fori_loop(0, NCH, chunk, 0)


def _gather_rows(queue_q, flat_idx):
    mesh = plsc.VectorSubcoreMesh(core_axis_name="c", subcore_axis_name="s")
    f = functools.partial(
        pl.kernel,
        mesh=mesh,
        out_type=jax.ShapeDtypeStruct((NB_ROWS, DIM), jnp.float32),
        scratch_types=[
            pltpu.VMEM((B_PER_W,), jnp.int32),
            pltpu.VMEM((CH, DIM), jnp.float32),
            pltpu.SemaphoreType.DMA,
        ],
    )(_gather_body)
    return f(queue_q, flat_idx)


def kernel(query, queue_q):
    qnorm = jnp.linalg.norm(query, ord=2, axis=-1, keepdims=True)
    tnorm = jnp.linalg.norm(queue_q, ord=2, axis=-1, keepdims=True)
    idx = _topk_indices(query, queue_q, qnorm, tnorm)
    flat_idx = idx[:, :TOP_N].reshape(-1)
    rows = _gather_rows(queue_q, flat_idx)
    return rows.reshape(B, TOP_N, DIM)
